# Initial kernel scaffold; baseline (speedup 1.0000x reference)
import jax
import jax.numpy as jnp
from jax.experimental import pallas as pl


def kernel(x, edge_index, batch, W1, b1, W2, b2):
    raise NotImplementedError("write your pallas kernel here")


# speedup vs baseline: 14.8501x; 14.8501x over previous
import dataclasses
import functools

import jax
import jax.numpy as jnp
from jax import lax
from jax.experimental import pallas as pl
from jax.experimental.pallas import tpu as pltpu
from jax.experimental.pallas import tpu_sc as plsc

N = 10000
E = 320000
G = 16
INC = 128
HIDC = 256
LATC = 128

NC = 2
NS = 16
NW = NC * NS
CHUNK = 128
JJ = 8
TT = 10
EPAD = NW * TT * JJ * CHUNK
NPAD = 10112
RPT = NPAD // NS

_MESH = plsc.VectorSubcoreMesh(core_axis_name="c", subcore_axis_name="s")

_SC_PARAMS = pltpu.CompilerParams()
if "needs_layout_passes" in pltpu.CompilerParams.__dataclass_fields__:
    _SC_PARAMS = dataclasses.replace(_SC_PARAMS, needs_layout_passes=False)


@functools.partial(
    pl.kernel,
    out_type=jax.ShapeDtypeStruct((NC, NPAD, 8), jnp.float32),
    mesh=_MESH,
    scratch_types=[
        pltpu.VMEM((JJ, CHUNK), jnp.int32),
        pltpu.VMEM((CHUNK, 8), jnp.float32),
        pltpu.VMEM_SHARED((NPAD, 8), jnp.float32),
    ],
)
def _deg_kernel(dst_hbm, ones_hbm, z8_hbm, out_hbm, dst_v, ones_v, deg_sh):
    c = lax.axis_index("c")
    s = lax.axis_index("s")
    wid = c * NS + s
    pltpu.sync_copy(ones_hbm, ones_v)
    pltpu.sync_copy(z8_hbm, deg_sh.at[pl.ds(s * RPT, RPT)])
    plsc.subcore_barrier()

    @pl.loop(0, TT)
    def _(t):
        pltpu.sync_copy(dst_hbm.at[wid, t], dst_v)

        @pl.loop(0, JJ)
        def _(j):
            pltpu.sync_copy(ones_v, deg_sh.at[dst_v.at[j]], add=True)

    plsc.subcore_barrier()
    pltpu.sync_copy(deg_sh.at[pl.ds(s * RPT, RPT)],
                    out_hbm.at[c, pl.ds(s * RPT, RPT)])


FIXM = (1 << 20) - 1


def _scale_body(deg_ref, x_ref, batch_ref, dinv_ref, xs_ref, vi_ref):
    dp = deg_ref[...]
    deg = dp[0, :, 0:1] + dp[1, :, 0:1] + 1.0
    row = lax.broadcasted_iota(jnp.int32, (NPAD, 1), 0)
    valid = row < N
    dinv = jnp.where(valid, lax.rsqrt(deg), 0.0)
    dinv_ref[...] = dinv
    xs_ref[...] = x_ref[...] * dinv
    fix = jnp.round(dinv * FIXM).astype(jnp.int32)
    packed = jnp.bitwise_or(jnp.left_shift(batch_ref[...], 20), fix)
    vi_ref[...] = jnp.where(valid, packed, 0)


@functools.partial(
    pl.kernel,
    out_type=jax.ShapeDtypeStruct((NC, NPAD, INC), jnp.float32),
    mesh=_MESH,
    scratch_types=[
        pltpu.VMEM((JJ, CHUNK), jnp.int32),
        pltpu.VMEM((JJ, CHUNK), jnp.int32),
        pltpu.VMEM((CHUNK, INC), jnp.float32),
        pltpu.VMEM_SHARED((NPAD, INC), jnp.float32),
    ],
)
def _prop_kernel(xs_hbm, src_hbm, dst_hbm, z128_hbm, p1_out,
                 src_v, dst_v, rows_v, p1_sh):
    c = lax.axis_index("c")
    s = lax.axis_index("s")
    wid = c * NS + s
    pltpu.sync_copy(z128_hbm, p1_sh.at[pl.ds(s * RPT, RPT)])
    plsc.subcore_barrier()

    @pl.loop(0, TT)
    def _(t):
        pltpu.sync_copy(src_hbm.at[wid, t], src_v)
        pltpu.sync_copy(dst_hbm.at[wid, t], dst_v)

        @pl.loop(0, JJ)
        def _(j):
            pltpu.sync_copy(xs_hbm.at[src_v.at[j]], rows_v)
            pltpu.sync_copy(rows_v, p1_sh.at[dst_v.at[j]], add=True)

    plsc.subcore_barrier()

    @pl.when(s == NS - 1)
    def _():
        pltpu.sync_copy(z128_hbm.at[pl.ds(0, 8)], p1_sh.at[pl.ds(N, 8)])

    plsc.subcore_barrier()
    pltpu.sync_copy(p1_sh.at[pl.ds(s * RPT, RPT)],
                    p1_out.at[c, pl.ds(s * RPT, RPT)])


@functools.partial(
    pl.kernel,
    out_type=jax.ShapeDtypeStruct((NC, NPAD, G), jnp.float32),
    mesh=_MESH,
    scratch_types=[
        pltpu.VMEM((JJ, CHUNK), jnp.int32),
        pltpu.VMEM((JJ, CHUNK), jnp.int32),
        pltpu.VMEM((CHUNK, G), jnp.float32),
        pltpu.VMEM((NPAD,), jnp.int32),
        pltpu.VMEM_SHARED((NPAD, G), jnp.float32),
    ],
    compiler_params=_SC_PARAMS,
)
def _wt_kernel(vi_hbm, src_hbm, dst_hbm, z16_hbm, wt_out,
               src_v, dst_v, oneh_v, vi_v, wt_sh):
    c = lax.axis_index("c")
    s = lax.axis_index("s")
    wid = c * NS + s
    pltpu.sync_copy(vi_hbm, vi_v)
    pltpu.sync_copy(z16_hbm, wt_sh.at[pl.ds(s * RPT, RPT)])
    pltpu.sync_copy(z16_hbm.at[pl.ds(0, CHUNK)], oneh_v)
    plsc.subcore_barrier()

    @pl.loop(0, TT)
    def _(t):
        pltpu.sync_copy(src_hbm.at[wid, t], src_v)
        pltpu.sync_copy(dst_hbm.at[wid, t], dst_v)

        @pl.loop(0, JJ)
        def _(j):
            zeros16 = jnp.zeros((16,), jnp.float32)
            saved = []
            for i in range(CHUNK // 16):
                d16 = dst_v[j, pl.ds(i * 16, 16)]
                v16 = plsc.load_gather(vi_v, [d16])
                g16 = lax.shift_right_logical(v16, 20)
                dd = v16.astype(jnp.float32) * (1.0 / FIXM)
                dd = dd - g16.astype(jnp.float32) * (float(1 << 20) / FIXM)
                row_i = lax.iota(jnp.int32, 16) + (i * 16)
                plsc.store_scatter(oneh_v, [row_i, g16], dd)
                saved.append((row_i, g16))
            pltpu.sync_copy(oneh_v, wt_sh.at[src_v.at[j]], add=True)
            for row_i, g16 in saved:
                plsc.store_scatter(oneh_v, [row_i, g16], zeros16)

    plsc.subcore_barrier()
    pltpu.sync_copy(wt_sh.at[pl.ds(s * RPT, RPT)],
                    wt_out.at[c, pl.ds(s * RPT, RPT)])


BLK = 2528
NBLK = NPAD // BLK


def _dense_body(p1_ref, xs_ref, dinv_ref, wt_ref, batch_ref,
                w1_ref, b1_ref, w2_ref, b2_ref, out_ref, acc, cacc):
    j = pl.program_id(0)

    @pl.when(j == 0)
    def _():
        acc[...] = jnp.zeros_like(acc)
        cacc[...] = jnp.zeros_like(cacc)

    hp = jax.lax.Precision.HIGHEST
    p1 = p1_ref[0] + p1_ref[1]
    dinv = dinv_ref[...]
    a1 = (p1 + xs_ref[...]) * dinv
    h = jnp.maximum(
        jnp.dot(a1, w1_ref[...], precision=hp) + b1_ref[...], 0.0)
    h2 = jnp.dot(h, w2_ref[...], precision=hp)

    bc = batch_ref[...]
    gi = lax.broadcasted_iota(jnp.int32, (BLK, G), 1)
    oneh = (gi == bc).astype(jnp.float32)
    wf = (wt_ref[0] + wt_ref[1] + oneh * dinv) * dinv
    tdims = (((0,), (0,)), ((), ()))
    acc[...] += lax.dot_general(wf, h2, tdims, precision=hp)
    cacc[...] += lax.dot_general(
        oneh, jnp.ones((BLK, LATC), jnp.float32), tdims, precision=hp)

    @pl.when(j == NBLK - 1)
    def _():
        out_ref[...] = acc[...] / jnp.maximum(cacc[...], 1.0) + b2_ref[...]


def kernel(x, edge_index, batch, W1, b1, W2, b2):
    src = edge_index[0].astype(jnp.int32)
    dst = edge_index[1].astype(jnp.int32)
    npad = EPAD - E
    src4 = jnp.concatenate([src, jnp.zeros((npad,), jnp.int32)])
    dst4 = jnp.concatenate([dst, jnp.full((npad,), N, jnp.int32)])
    src4 = src4.reshape(NW, TT, JJ, CHUNK)
    dst4 = dst4.reshape(NW, TT, JJ, CHUNK)
    batch_i = batch.astype(jnp.int32)
    batch_col = jnp.concatenate(
        [batch_i, jnp.full((NPAD - N,), -1, jnp.int32)]).reshape(NPAD, 1)
    x_p = jnp.concatenate(
        [x, jnp.zeros((NPAD - N, INC), jnp.float32)], axis=0)

    ones8 = jnp.ones((CHUNK, 8), jnp.float32)
    z8 = jnp.zeros((RPT, 8), jnp.float32)
    z16 = jnp.zeros((RPT, G), jnp.float32)
    z128 = jnp.zeros((RPT, INC), jnp.float32)

    deg_parts = _deg_kernel(dst4, ones8, z8)

    dinv_col, xs, vi = pl.pallas_call(
        _scale_body,
        out_shape=[
            jax.ShapeDtypeStruct((NPAD, 1), jnp.float32),
            jax.ShapeDtypeStruct((NPAD, INC), jnp.float32),
            jax.ShapeDtypeStruct((NPAD, 1), jnp.int32),
        ],
    )(deg_parts, x_p, batch_col)

    p1_parts = _prop_kernel(xs, src4, dst4, z128)
    wt_parts = _wt_kernel(vi.reshape(NPAD), src4, dst4, z16)

    latent = pl.pallas_call(
        _dense_body,
        grid=(NBLK,),
        in_specs=[
            pl.BlockSpec((NC, BLK, INC), lambda j: (0, j, 0)),
            pl.BlockSpec((BLK, INC), lambda j: (j, 0)),
            pl.BlockSpec((BLK, 1), lambda j: (j, 0)),
            pl.BlockSpec((NC, BLK, G), lambda j: (0, j, 0)),
            pl.BlockSpec((BLK, 1), lambda j: (j, 0)),
            pl.BlockSpec((INC, HIDC), lambda j: (0, 0)),
            pl.BlockSpec((1, HIDC), lambda j: (0, 0)),
            pl.BlockSpec((HIDC, LATC), lambda j: (0, 0)),
            pl.BlockSpec((1, LATC), lambda j: (0, 0)),
        ],
        out_specs=pl.BlockSpec((G, LATC), lambda j: (0, 0)),
        out_shape=jax.ShapeDtypeStruct((G, LATC), jnp.float32),
        scratch_shapes=[
            pltpu.VMEM((G, LATC), jnp.float32),
            pltpu.VMEM((G, LATC), jnp.float32),
        ],
    )(p1_parts, xs, dinv_col, wt_parts, batch_col,
      W1, b1.reshape(1, HIDC), W2, b2.reshape(1, LATC))

    return latent


# speedup vs baseline: 15.6322x; 1.0527x over previous
import dataclasses
import functools

import jax
import jax.numpy as jnp
from jax import lax
from jax.experimental import pallas as pl
from jax.experimental.pallas import tpu as pltpu
from jax.experimental.pallas import tpu_sc as plsc

N = 10000
E = 320000
G = 16
INC = 128
HIDC = 256
LATC = 128

NC = 2
NS = 16
NW = NC * NS
CHUNK = 128
JJ = 8
TT = 10
EPAD = NW * TT * JJ * CHUNK
NPAD = 10112
RPT = NPAD // NS

_MESH = plsc.VectorSubcoreMesh(core_axis_name="c", subcore_axis_name="s")

_SC_PARAMS = pltpu.CompilerParams()
if "needs_layout_passes" in pltpu.CompilerParams.__dataclass_fields__:
    _SC_PARAMS = dataclasses.replace(_SC_PARAMS, needs_layout_passes=False)


@functools.partial(
    pl.kernel,
    out_type=jax.ShapeDtypeStruct((NC, NPAD, 8), jnp.float32),
    mesh=_MESH,
    scratch_types=[
        pltpu.VMEM((JJ, CHUNK), jnp.int32),
        pltpu.VMEM((CHUNK, 8), jnp.float32),
        pltpu.VMEM_SHARED((NPAD, 8), jnp.float32),
    ],
)
def _deg_kernel(dst_hbm, ones_hbm, z8_hbm, out_hbm, dst_v, ones_v, deg_sh):
    c = lax.axis_index("c")
    s = lax.axis_index("s")
    wid = c * NS + s
    pltpu.sync_copy(ones_hbm, ones_v)
    pltpu.sync_copy(z8_hbm, deg_sh.at[pl.ds(s * RPT, RPT)])
    plsc.subcore_barrier()

    @pl.loop(0, TT)
    def _(t):
        pltpu.sync_copy(dst_hbm.at[wid, t], dst_v)

        @pl.loop(0, JJ)
        def _(j):
            pltpu.sync_copy(ones_v, deg_sh.at[dst_v.at[j]], add=True)

    plsc.subcore_barrier()
    pltpu.sync_copy(deg_sh.at[pl.ds(s * RPT, RPT)],
                    out_hbm.at[c, pl.ds(s * RPT, RPT)])


FIXM = (1 << 20) - 1


def _scale_body(deg_ref, x_ref, batch_ref, dinv_ref, xs_ref, vi_ref):
    dp = deg_ref[...]
    deg = dp[0, :, 0:1] + dp[1, :, 0:1] + 1.0
    row = lax.broadcasted_iota(jnp.int32, (NPAD, 1), 0)
    valid = row < N
    dinv = jnp.where(valid, lax.rsqrt(deg), 0.0)
    dinv_ref[...] = dinv
    xs_ref[...] = x_ref[...] * dinv
    fix = jnp.round(dinv * FIXM).astype(jnp.int32)
    packed = jnp.bitwise_or(jnp.left_shift(batch_ref[...], 20), fix)
    vi_ref[...] = jnp.where(valid, packed, 0)


@functools.partial(
    pl.kernel,
    out_type=jax.ShapeDtypeStruct((NC, NPAD, INC), jnp.float32),
    mesh=_MESH,
    scratch_types=[
        pltpu.VMEM((JJ, CHUNK), jnp.int32),
        pltpu.VMEM((JJ, CHUNK), jnp.int32),
        pltpu.VMEM((CHUNK, INC), jnp.float32),
        pltpu.VMEM((CHUNK, INC), jnp.float32),
        pltpu.SemaphoreType.DMA,
        pltpu.SemaphoreType.DMA,
        pltpu.VMEM_SHARED((NPAD, INC), jnp.float32),
    ],
)
def _prop_kernel(xs_hbm, src_hbm, dst_hbm, z128_hbm, p1_out,
                 src_v, dst_v, rows_a, rows_b, gsem, ssem, p1_sh):
    c = lax.axis_index("c")
    s = lax.axis_index("s")
    wid = c * NS + s
    pltpu.sync_copy(z128_hbm, p1_sh.at[pl.ds(s * RPT, RPT)])
    plsc.subcore_barrier()

    bufs = (rows_a, rows_b)

    @pl.loop(0, TT)
    def _(t):
        pltpu.sync_copy(src_hbm.at[wid, t], src_v)
        pltpu.sync_copy(dst_hbm.at[wid, t], dst_v)
        gd = pltpu.async_copy(xs_hbm.at[src_v.at[0]], bufs[0], gsem)
        for j in range(JJ):
            gd.wait()
            if j < JJ - 1:
                gd = pltpu.async_copy(
                    xs_hbm.at[src_v.at[j + 1]], bufs[(j + 1) % 2], gsem)
            pltpu.sync_copy(bufs[j % 2], p1_sh.at[dst_v.at[j]], add=True)

    plsc.subcore_barrier()

    @pl.when(s == NS - 1)
    def _():
        pltpu.sync_copy(z128_hbm.at[pl.ds(0, 8)], p1_sh.at[pl.ds(N, 8)])

    plsc.subcore_barrier()
    pltpu.sync_copy(p1_sh.at[pl.ds(s * RPT, RPT)],
                    p1_out.at[c, pl.ds(s * RPT, RPT)])


@functools.partial(
    pl.kernel,
    out_type=jax.ShapeDtypeStruct((NC, NPAD, G), jnp.float32),
    mesh=_MESH,
    scratch_types=[
        pltpu.VMEM((JJ, CHUNK), jnp.int32),
        pltpu.VMEM((JJ, CHUNK), jnp.int32),
        pltpu.VMEM((CHUNK, G), jnp.float32),
        pltpu.VMEM((NPAD,), jnp.int32),
        pltpu.VMEM_SHARED((NPAD, G), jnp.float32),
    ],
    compiler_params=_SC_PARAMS,
)
def _wt_kernel(vi_hbm, src_hbm, dst_hbm, z16_hbm, wt_out,
               src_v, dst_v, oneh_v, vi_v, wt_sh):
    c = lax.axis_index("c")
    s = lax.axis_index("s")
    wid = c * NS + s
    pltpu.sync_copy(vi_hbm, vi_v)
    pltpu.sync_copy(z16_hbm, wt_sh.at[pl.ds(s * RPT, RPT)])
    pltpu.sync_copy(z16_hbm.at[pl.ds(0, CHUNK)], oneh_v)
    plsc.subcore_barrier()

    @pl.loop(0, TT)
    def _(t):
        pltpu.sync_copy(src_hbm.at[wid, t], src_v)
        pltpu.sync_copy(dst_hbm.at[wid, t], dst_v)

        @pl.loop(0, JJ)
        def _(j):
            zeros16 = jnp.zeros((16,), jnp.float32)
            saved = []
            for i in range(CHUNK // 16):
                d16 = dst_v[j, pl.ds(i * 16, 16)]
                v16 = plsc.load_gather(vi_v, [d16])
                g16 = lax.shift_right_logical(v16, 20)
                dd = v16.astype(jnp.float32) * (1.0 / FIXM)
                dd = dd - g16.astype(jnp.float32) * (float(1 << 20) / FIXM)
                row_i = lax.iota(jnp.int32, 16) + (i * 16)
                plsc.store_scatter(oneh_v, [row_i, g16], dd)
                saved.append((row_i, g16))
            pltpu.sync_copy(oneh_v, wt_sh.at[src_v.at[j]], add=True)
            for row_i, g16 in saved:
                plsc.store_scatter(oneh_v, [row_i, g16], zeros16)

    plsc.subcore_barrier()
    pltpu.sync_copy(wt_sh.at[pl.ds(s * RPT, RPT)],
                    wt_out.at[c, pl.ds(s * RPT, RPT)])


BLK = 2528
NBLK = NPAD // BLK


def _dense_body(p1_ref, xs_ref, dinv_ref, wt_ref, batch_ref,
                w1_ref, b1_ref, w2_ref, b2_ref, out_ref, acc, cacc):
    j = pl.program_id(0)

    @pl.when(j == 0)
    def _():
        acc[...] = jnp.zeros_like(acc)
        cacc[...] = jnp.zeros_like(cacc)

    hp = jax.lax.Precision.HIGHEST
    p1 = p1_ref[0] + p1_ref[1]
    dinv = dinv_ref[...]
    a1 = (p1 + xs_ref[...]) * dinv
    h = jnp.maximum(
        jnp.dot(a1, w1_ref[...], precision=hp) + b1_ref[...], 0.0)
    h2 = jnp.dot(h, w2_ref[...], precision=hp)

    bc = batch_ref[...]
    gi = lax.broadcasted_iota(jnp.int32, (BLK, G), 1)
    oneh = (gi == bc).astype(jnp.float32)
    wf = (wt_ref[0] + wt_ref[1] + oneh * dinv) * dinv
    tdims = (((0,), (0,)), ((), ()))
    acc[...] += lax.dot_general(wf, h2, tdims, precision=hp)
    cacc[...] += lax.dot_general(
        oneh, jnp.ones((BLK, LATC), jnp.float32), tdims, precision=hp)

    @pl.when(j == NBLK - 1)
    def _():
        out_ref[...] = acc[...] / jnp.maximum(cacc[...], 1.0) + b2_ref[...]


def kernel(x, edge_index, batch, W1, b1, W2, b2):
    src = edge_index[0].astype(jnp.int32)
    dst = edge_index[1].astype(jnp.int32)
    npad = EPAD - E
    src4 = jnp.concatenate([src, jnp.zeros((npad,), jnp.int32)])
    dst4 = jnp.concatenate([dst, jnp.full((npad,), N, jnp.int32)])
    src4 = src4.reshape(NW, TT, JJ, CHUNK)
    dst4 = dst4.reshape(NW, TT, JJ, CHUNK)
    batch_i = batch.astype(jnp.int32)
    batch_col = jnp.concatenate(
        [batch_i, jnp.full((NPAD - N,), -1, jnp.int32)]).reshape(NPAD, 1)
    x_p = jnp.concatenate(
        [x, jnp.zeros((NPAD - N, INC), jnp.float32)], axis=0)

    ones8 = jnp.ones((CHUNK, 8), jnp.float32)
    z8 = jnp.zeros((RPT, 8), jnp.float32)
    z16 = jnp.zeros((RPT, G), jnp.float32)
    z128 = jnp.zeros((RPT, INC), jnp.float32)

    deg_parts = _deg_kernel(dst4, ones8, z8)

    dinv_col, xs, vi = pl.pallas_call(
        _scale_body,
        out_shape=[
            jax.ShapeDtypeStruct((NPAD, 1), jnp.float32),
            jax.ShapeDtypeStruct((NPAD, INC), jnp.float32),
            jax.ShapeDtypeStruct((NPAD, 1), jnp.int32),
        ],
    )(deg_parts, x_p, batch_col)

    p1_parts = _prop_kernel(xs, src4, dst4, z128)
    wt_parts = _wt_kernel(vi.reshape(NPAD), src4, dst4, z16)

    latent = pl.pallas_call(
        _dense_body,
        grid=(NBLK,),
        in_specs=[
            pl.BlockSpec((NC, BLK, INC), lambda j: (0, j, 0)),
            pl.BlockSpec((BLK, INC), lambda j: (j, 0)),
            pl.BlockSpec((BLK, 1), lambda j: (j, 0)),
            pl.BlockSpec((NC, BLK, G), lambda j: (0, j, 0)),
            pl.BlockSpec((BLK, 1), lambda j: (j, 0)),
            pl.BlockSpec((INC, HIDC), lambda j: (0, 0)),
            pl.BlockSpec((1, HIDC), lambda j: (0, 0)),
            pl.BlockSpec((HIDC, LATC), lambda j: (0, 0)),
            pl.BlockSpec((1, LATC), lambda j: (0, 0)),
        ],
        out_specs=pl.BlockSpec((G, LATC), lambda j: (0, 0)),
        out_shape=jax.ShapeDtypeStruct((G, LATC), jnp.float32),
        scratch_shapes=[
            pltpu.VMEM((G, LATC), jnp.float32),
            pltpu.VMEM((G, LATC), jnp.float32),
        ],
    )(p1_parts, xs, dinv_col, wt_parts, batch_col,
      W1, b1.reshape(1, HIDC), W2, b2.reshape(1, LATC))

    return latent
